# baseline (speedup 1.0000x reference)
import functools

import jax
import jax.numpy as jnp
from jax import lax
from jax.experimental import pallas as pl
from jax.experimental.pallas import tpu as pltpu
from jax.experimental.pallas import tpu_sc as plsc

NC = 2
NS = 16
LN = 16
NW = NC * NS

SUB = 128
CHB = 4
CH = SUB * CHB
NBUF = 3
PITCH = 128
WDP = 64


def _build(N, D, WD, PD, PV, per_w, n_chunks):
  mesh = plsc.VectorSubcoreMesh(
      core_axis_name="c", subcore_axis_name="s", num_cores=NC,
      num_subcores=NS)

  @functools.partial(
      pl.kernel,
      out_type=jax.ShapeDtypeStruct((N, WDP), jnp.float32),
      mesh=mesh,
      compiler_params=pltpu.CompilerParams(
          needs_layout_passes=False, use_tc_tiling_on_sc=False),
      scratch_types=[
          [pltpu.VMEM((CH,), jnp.int32)] * NBUF,
          [pltpu.VMEM((CH,), jnp.int32)] * NBUF,
          [pltpu.VMEM((CH,), jnp.int32)] * NBUF,
          [pltpu.VMEM((CH, WDP), jnp.float32)] * NBUF,
          pltpu.VMEM((PV * PD,), jnp.float32),
          pltpu.VMEM((PV * PD,), jnp.float32),
          [pltpu.SemaphoreType.DMA] * NBUF,
          [pltpu.SemaphoreType.DMA] * NBUF,
          [pltpu.SemaphoreType.DMA] * NBUF,
      ],
  )
  def k(wf_h, p1_h, p2_h, wt_h, p1t_h, p2t_h, out_h,
        widx, p1i, p2i, outv, p1t, p2t, gsem, wsem, isem):
    wid = lax.axis_index("s") * NC + lax.axis_index("c")
    base = wid * per_w
    pltpu.sync_copy(p1t_h, p1t)
    pltpu.sync_copy(p2t_h, p2t)
    lane = lax.iota(jnp.int32, 16)

    def fire_idx(c, b):
      row0 = base + c * CH
      pltpu.async_copy(wf_h.at[pl.ds(row0, CH)], widx[b], isem[b])
      pltpu.async_copy(p1_h.at[pl.ds(row0, CH)], p1i[b], isem[b])
      pltpu.async_copy(p2_h.at[pl.ds(row0, CH)], p2i[b], isem[b])

    def wait_idx(b):
      pltpu.make_async_copy(wf_h.at[pl.ds(0, CH)], widx[b], isem[b]).wait()
      pltpu.make_async_copy(wf_h.at[pl.ds(0, CH)], p1i[b], isem[b]).wait()
      pltpu.make_async_copy(wf_h.at[pl.ds(0, CH)], p2i[b], isem[b]).wait()

    def fire_gathers(b):
      for bb in range(CHB):
        pltpu.async_copy(
            wt_h.at[widx[b].at[pl.ds(bb * SUB, SUB)]],
            outv[b].at[pl.ds(bb * SUB, SUB), :], gsem[b])

    def drain_gathers(b):
      pltpu.make_async_copy(
          wt_h.at[pl.ds(0, CH), :], outv[b], gsem[b]).wait()

    def fire_writeout(c, b):
      row0 = base + c * CH
      pltpu.async_copy(
          outv[b], out_h.at[pl.ds(row0, CH), :], wsem[b])

    def drain_writeout(b):
      pltpu.make_async_copy(
          out_h.at[pl.ds(0, CH), :], outv[b], wsem[b]).wait()

    def pos_compute(b):
      @pl.loop(0, CH // LN)
      def _pos(g):
        rows = g * LN + lane
        i1 = p1i[b][pl.ds(g * LN, LN)] * PD
        i2 = p2i[b][pl.ds(g * LN, LN)] * PD
        for j in range(PD):
          v1 = plsc.load_gather(p1t, [i1 + j])
          plsc.store_scatter(
              outv[b], [rows, jnp.full((LN,), WD + j, jnp.int32)], v1)
          v2 = plsc.load_gather(p2t, [i2 + j])
          plsc.store_scatter(
              outv[b], [rows, jnp.full((LN,), WD + PD + j, jnp.int32)], v2)

    for c0 in range(NBUF):
      fire_idx(c0, c0)
    for c0 in range(NBUF - 1):
      wait_idx(c0)
      fire_gathers(c0)

    n_main = (n_chunks // NBUF) * NBUF

    def step(c, b):
      drain_gathers(b)
      pos_compute(b)
      fire_writeout(c, b)

      @pl.when(c + NBUF < n_chunks)
      def _():
        fire_idx(c + NBUF, b)

      b2 = (b + NBUF - 1) % NBUF

      @pl.when(c + NBUF - 1 < n_chunks)
      def _():
        @pl.when(c >= 1)
        def _():
          drain_writeout(b2)

        wait_idx(b2)
        fire_gathers(b2)

    @pl.loop(0, n_main, step=NBUF)
    def _outer(g):
      for b in range(NBUF):
        step(g + b, b)

    for c in range(n_main, n_chunks):
      step(c, c % NBUF)

    for c in range(n_chunks - NBUF, n_chunks):
      drain_writeout(c % NBUF)

  return k


def kernel(word, pos1, pos2, word_table, pos1_table, pos2_table):
  B, L = word.shape
  V, WD = word_table.shape
  PV, PD = pos1_table.shape
  D = WD + 2 * PD
  N = B * L
  assert N % (NW * CH) == 0
  per_w = N // NW
  n_chunks = per_w // CH
  assert n_chunks >= NBUF

  wf = word.reshape(N).astype(jnp.int32)
  p1f = pos1.reshape(N).astype(jnp.int32)
  p2f = pos2.reshape(N).astype(jnp.int32)
  p1t = pos1_table.reshape(PV * PD)
  p2t = pos2_table.reshape(PV * PD)

  wt_pad = jnp.pad(word_table, ((0, 0), (0, WDP - WD)))

  k = _build(N, D, WD, PD, PV, per_w, n_chunks)
  out = k(wf, p1f, p2f, wt_pad, p1t, p2t)
  return out.reshape(B, L, WDP)[:, :, :D]


# speedup vs baseline: 1.6296x; 1.6296x over previous
import functools

import jax
import jax.numpy as jnp
from jax import lax
from jax.experimental import pallas as pl
from jax.experimental.pallas import tpu as pltpu
from jax.experimental.pallas import tpu_sc as plsc

NC = 2
NS = 16
LN = 16
NW = NC * NS

SUB = 128
CHB = 4
CH = SUB * CHB
NBUF = 3
PITCH = 128
WDP = 64


def _build(N, D, WD, PD, PV, per_w, n_chunks):
  mesh = plsc.VectorSubcoreMesh(
      core_axis_name="c", subcore_axis_name="s", num_cores=NC,
      num_subcores=NS)

  @functools.partial(
      pl.kernel,
      out_type=jax.ShapeDtypeStruct((N, PITCH), jnp.float32),
      mesh=mesh,
      compiler_params=pltpu.CompilerParams(
          needs_layout_passes=False, use_tc_tiling_on_sc=False),
      scratch_types=[
          [pltpu.VMEM((CH,), jnp.int32)] * NBUF,
          [pltpu.VMEM((CH,), jnp.int32)] * NBUF,
          [pltpu.VMEM((CH,), jnp.int32)] * NBUF,
          [pltpu.VMEM((CH, WDP), jnp.float32)] * NBUF,
          pltpu.VMEM((PV * PD,), jnp.float32),
          pltpu.VMEM((PV * PD,), jnp.float32),
          [pltpu.SemaphoreType.DMA] * NBUF,
          [pltpu.SemaphoreType.DMA] * NBUF,
          [pltpu.SemaphoreType.DMA] * NBUF,
      ],
  )
  def k(wf_h, p1_h, p2_h, wt_h, p1t_h, p2t_h, out_h,
        widx, p1i, p2i, outv, p1t, p2t, gsem, wsem, isem):
    wid = lax.axis_index("s") * NC + lax.axis_index("c")
    base = wid * per_w
    pltpu.sync_copy(p1t_h, p1t)
    pltpu.sync_copy(p2t_h, p2t)
    lane = lax.iota(jnp.int32, 16)

    def fire_idx(c, b):
      row0 = base + c * CH
      pltpu.async_copy(wf_h.at[pl.ds(row0, CH)], widx[b], isem[b])
      pltpu.async_copy(p1_h.at[pl.ds(row0, CH)], p1i[b], isem[b])
      pltpu.async_copy(p2_h.at[pl.ds(row0, CH)], p2i[b], isem[b])

    def wait_idx(b):
      pltpu.make_async_copy(wf_h.at[pl.ds(0, CH)], widx[b], isem[b]).wait()
      pltpu.make_async_copy(wf_h.at[pl.ds(0, CH)], p1i[b], isem[b]).wait()
      pltpu.make_async_copy(wf_h.at[pl.ds(0, CH)], p2i[b], isem[b]).wait()

    def fire_gathers(b):
      for bb in range(CHB):
        pltpu.async_copy(
            wt_h.at[widx[b].at[pl.ds(bb * SUB, SUB)]],
            outv[b].at[pl.ds(bb * SUB, SUB), :], gsem[b])

    def drain_gathers(b):
      pltpu.make_async_copy(
          wt_h.at[pl.ds(0, CH), :], outv[b], gsem[b]).wait()

    def fire_writeout(c, b):
      row0 = base + c * CH
      pltpu.async_copy(
          outv[b], out_h.at[pl.ds(row0, CH), pl.ds(0, WDP)], wsem[b])

    def drain_writeout(b):
      pltpu.make_async_copy(
          out_h.at[pl.ds(0, CH), pl.ds(0, WDP)], outv[b], wsem[b]).wait()

    def pos_compute(b):
      @pl.loop(0, CH // LN)
      def _pos(g):
        rows = g * LN + lane
        i1 = p1i[b][pl.ds(g * LN, LN)] * PD
        i2 = p2i[b][pl.ds(g * LN, LN)] * PD
        for j in range(PD):
          v1 = plsc.load_gather(p1t, [i1 + j])
          plsc.store_scatter(
              outv[b], [rows, jnp.full((LN,), WD + j, jnp.int32)], v1)
          v2 = plsc.load_gather(p2t, [i2 + j])
          plsc.store_scatter(
              outv[b], [rows, jnp.full((LN,), WD + PD + j, jnp.int32)], v2)

    for c0 in range(NBUF):
      fire_idx(c0, c0)
    for c0 in range(NBUF - 1):
      wait_idx(c0)
      fire_gathers(c0)

    n_main = (n_chunks // NBUF) * NBUF

    def step(c, b):
      drain_gathers(b)
      pos_compute(b)
      fire_writeout(c, b)

      @pl.when(c + NBUF < n_chunks)
      def _():
        fire_idx(c + NBUF, b)

      b2 = (b + NBUF - 1) % NBUF

      @pl.when(c + NBUF - 1 < n_chunks)
      def _():
        @pl.when(c >= 1)
        def _():
          drain_writeout(b2)

        wait_idx(b2)
        fire_gathers(b2)

    @pl.loop(0, n_main, step=NBUF)
    def _outer(g):
      for b in range(NBUF):
        step(g + b, b)

    for c in range(n_main, n_chunks):
      step(c, c % NBUF)

    for c in range(n_chunks - NBUF, n_chunks):
      drain_writeout(c % NBUF)

  return k


def kernel(word, pos1, pos2, word_table, pos1_table, pos2_table):
  B, L = word.shape
  V, WD = word_table.shape
  PV, PD = pos1_table.shape
  D = WD + 2 * PD
  N = B * L
  assert N % (NW * CH) == 0
  per_w = N // NW
  n_chunks = per_w // CH
  assert n_chunks >= NBUF

  wf = word.reshape(N).astype(jnp.int32)
  p1f = pos1.reshape(N).astype(jnp.int32)
  p2f = pos2.reshape(N).astype(jnp.int32)
  p1t = pos1_table.reshape(PV * PD)
  p2t = pos2_table.reshape(PV * PD)

  wt_pad = jnp.pad(word_table, ((0, 0), (0, WDP - WD)))

  k = _build(N, D, WD, PD, PV, per_w, n_chunks)
  out = k(wf, p1f, p2f, wt_pad, p1t, p2t)
  return out.reshape(B, L, PITCH)[:, :, :D]


# speedup vs baseline: 1.6334x; 1.0023x over previous
import functools

import jax
import jax.numpy as jnp
from jax import lax
from jax.experimental import pallas as pl
from jax.experimental.pallas import tpu as pltpu
from jax.experimental.pallas import tpu_sc as plsc

NC = 2
NS = 16
LN = 16
NW = NC * NS

SUB = 128
CHB = 2
CH = SUB * CHB
NBUF = 4
PITCH = 128
WDP = 64


def _build(N, D, WD, PD, PV, per_w, n_chunks):
  mesh = plsc.VectorSubcoreMesh(
      core_axis_name="c", subcore_axis_name="s", num_cores=NC,
      num_subcores=NS)

  @functools.partial(
      pl.kernel,
      out_type=jax.ShapeDtypeStruct((N, PITCH), jnp.float32),
      mesh=mesh,
      compiler_params=pltpu.CompilerParams(
          needs_layout_passes=False, use_tc_tiling_on_sc=False),
      scratch_types=[
          [pltpu.VMEM((CH,), jnp.int32)] * NBUF,
          [pltpu.VMEM((CH,), jnp.int32)] * NBUF,
          [pltpu.VMEM((CH,), jnp.int32)] * NBUF,
          [pltpu.VMEM((CH, WDP), jnp.float32)] * NBUF,
          pltpu.VMEM((PV * PD,), jnp.float32),
          pltpu.VMEM((PV * PD,), jnp.float32),
          [pltpu.SemaphoreType.DMA] * NBUF,
          [pltpu.SemaphoreType.DMA] * NBUF,
          [pltpu.SemaphoreType.DMA] * NBUF,
      ],
  )
  def k(wf_h, p1_h, p2_h, wt_h, p1t_h, p2t_h, out_h,
        widx, p1i, p2i, outv, p1t, p2t, gsem, wsem, isem):
    wid = lax.axis_index("s") * NC + lax.axis_index("c")
    base = wid * per_w
    pltpu.sync_copy(p1t_h, p1t)
    pltpu.sync_copy(p2t_h, p2t)
    lane = lax.iota(jnp.int32, 16)

    def fire_idx(c, b):
      row0 = base + c * CH
      pltpu.async_copy(wf_h.at[pl.ds(row0, CH)], widx[b], isem[b])
      pltpu.async_copy(p1_h.at[pl.ds(row0, CH)], p1i[b], isem[b])
      pltpu.async_copy(p2_h.at[pl.ds(row0, CH)], p2i[b], isem[b])

    def wait_idx(b):
      pltpu.make_async_copy(wf_h.at[pl.ds(0, CH)], widx[b], isem[b]).wait()
      pltpu.make_async_copy(wf_h.at[pl.ds(0, CH)], p1i[b], isem[b]).wait()
      pltpu.make_async_copy(wf_h.at[pl.ds(0, CH)], p2i[b], isem[b]).wait()

    def fire_gathers(b):
      for bb in range(CHB):
        pltpu.async_copy(
            wt_h.at[widx[b].at[pl.ds(bb * SUB, SUB)]],
            outv[b].at[pl.ds(bb * SUB, SUB), :], gsem[b])

    def drain_gathers(b):
      pltpu.make_async_copy(
          wt_h.at[pl.ds(0, CH), :], outv[b], gsem[b]).wait()

    def fire_writeout(c, b):
      row0 = base + c * CH
      pltpu.async_copy(
          outv[b], out_h.at[pl.ds(row0, CH), pl.ds(0, WDP)], wsem[b])

    def drain_writeout(b):
      pltpu.make_async_copy(
          out_h.at[pl.ds(0, CH), pl.ds(0, WDP)], outv[b], wsem[b]).wait()

    def pos_compute(b):
      @pl.loop(0, CH // LN)
      def _pos(g):
        rows = g * LN + lane
        i1 = p1i[b][pl.ds(g * LN, LN)] * PD
        i2 = p2i[b][pl.ds(g * LN, LN)] * PD
        for j in range(PD):
          v1 = plsc.load_gather(p1t, [i1 + j])
          plsc.store_scatter(
              outv[b], [rows, jnp.full((LN,), WD + j, jnp.int32)], v1)
          v2 = plsc.load_gather(p2t, [i2 + j])
          plsc.store_scatter(
              outv[b], [rows, jnp.full((LN,), WD + PD + j, jnp.int32)], v2)

    for c0 in range(NBUF):
      fire_idx(c0, c0)
    for c0 in range(NBUF - 1):
      wait_idx(c0)
      fire_gathers(c0)

    n_main = (n_chunks // NBUF) * NBUF

    def step(c, b):
      drain_gathers(b)
      pos_compute(b)
      fire_writeout(c, b)

      @pl.when(c + NBUF < n_chunks)
      def _():
        fire_idx(c + NBUF, b)

      b2 = (b + NBUF - 1) % NBUF

      @pl.when(c + NBUF - 1 < n_chunks)
      def _():
        @pl.when(c >= 1)
        def _():
          drain_writeout(b2)

        wait_idx(b2)
        fire_gathers(b2)

    @pl.loop(0, n_main, step=NBUF)
    def _outer(g):
      for b in range(NBUF):
        step(g + b, b)

    for c in range(n_main, n_chunks):
      step(c, c % NBUF)

    for c in range(n_chunks - NBUF, n_chunks):
      drain_writeout(c % NBUF)

  return k


def kernel(word, pos1, pos2, word_table, pos1_table, pos2_table):
  B, L = word.shape
  V, WD = word_table.shape
  PV, PD = pos1_table.shape
  D = WD + 2 * PD
  N = B * L
  assert N % (NW * CH) == 0
  per_w = N // NW
  n_chunks = per_w // CH
  assert n_chunks >= NBUF

  wf = word.reshape(N).astype(jnp.int32)
  p1f = pos1.reshape(N).astype(jnp.int32)
  p2f = pos2.reshape(N).astype(jnp.int32)
  p1t = pos1_table.reshape(PV * PD)
  p2t = pos2_table.reshape(PV * PD)

  wt_pad = jnp.pad(word_table, ((0, 0), (0, WDP - WD)))

  k = _build(N, D, WD, PD, PV, per_w, n_chunks)
  out = k(wf, p1f, p2f, wt_pad, p1t, p2t)
  return out.reshape(B, L, PITCH)[:, :, :D]
